# baseline (speedup 1.0000x reference)
import functools

import jax
import jax.numpy as jnp
from jax import lax
from jax.experimental import pallas as pl
from jax.experimental.pallas import tpu as pltpu
from jax.experimental.pallas import tpu_sc as plsc

N = 50000
E = 500000
D = 128
L = 5

NTILE = 16
NW = 32

NCHUNK = 6
CH = 8448
NPAD = NCHUNK * CH
DUMP = 256
ACC_ROWS = CH + DUMP
ZSPAN = ACC_ROWS // NTILE
ZB = 16
DROWS = CH // NTILE

E_PAD = 524288
EPT = E_PAD // NW
SEG = 2048
NSEG = EPT // SEG
BCAP = 16896
BINS_LEN = 2 * NCHUNK * NW * BCAP
CNTS_LEN = 2 * NW * 16

_GDN = lax.GatherDimensionNumbers(
    offset_dims=(), collapsed_slice_dims=(0,), start_index_map=(0,))


def _lgather(x, idx):
    return lax.gather(x, idx[:, None], _GDN, slice_sizes=(1,),
                      mode=lax.GatherScatterMode.PROMISE_IN_BOUNDS)


def _zero_vmem(ref, rows, width):
    zero16 = jnp.zeros((16,), jnp.float32)

    def body(i, _):
        for k in range(width // 16):
            ref[i, pl.ds(k * 16, 16)] = zero16
        return 0
    lax.fori_loop(0, rows, body, 0)


def _sc_binner():
    out_types = (jax.ShapeDtypeStruct((BINS_LEN,), jnp.int32),
                 jax.ShapeDtypeStruct((CNTS_LEN,), jnp.int32))
    scratch = (
        pltpu.VMEM((SEG,), jnp.int32),
        pltpu.VMEM((SEG,), jnp.int32),
        tuple(pltpu.VMEM((BCAP,), jnp.int32) for _ in range(NCHUNK)),
        pltpu.VMEM((16,), jnp.int32),
    )
    mesh = plsc.VectorSubcoreMesh(core_axis_name="c", subcore_axis_name="s")

    @functools.partial(pl.kernel, out_type=out_types, mesh=mesh,
                       scratch_types=scratch)
    def binner(src_w, dst_w, src_b, dst_b, bins, counts, sbuf, dbuf,
               pends, cvbuf):
        scid = lax.axis_index("c")
        sid = lax.axis_index("s")
        w = scid * NTILE + sid
        lane = lax.iota(jnp.int32, 16)

        for t in range(2):
            src_e = (src_w, src_b)[t]
            dst_e = (dst_w, dst_b)[t]

            def seg_body(seg, cnts):
                base = w * EPT + seg * SEG
                pltpu.sync_copy(src_e.at[pl.ds(base, SEG)], sbuf)
                pltpu.sync_copy(dst_e.at[pl.ds(base, SEG)], dbuf)

                def vec_body(i, cnts):
                    s = sbuf[pl.ds(i * 16, 16)]
                    d = dbuf[pl.ds(i * 16, 16)]
                    cid = jnp.zeros((16,), jnp.int32)
                    for c in range(1, NCHUNK + 1):
                        cid = jnp.where(d >= c * CH, cid + 1, cid)
                    comb = s * 16384 + (d - cid * CH)
                    mi = lax.shift_left(jnp.int32(1),
                                        jnp.minimum(cid, 6) * 5)
                    cs = mi
                    for k in (1, 2, 4, 8):
                        sh = _lgather(cs, jnp.maximum(lane - k, 0))
                        cs = cs + jnp.where(lane >= k, sh, 0)
                    cs15 = cs[15]
                    new = []
                    for c in range(NCHUNK):
                        m = cid == c
                        em = jnp.where(m, 1, 0)
                        ec = ((cs >> (5 * c)) & 31) - em
                        x = comb
                        dd = jnp.where(m, lane - ec, 0)
                        mm = em
                        for b in (1, 2, 4, 8):
                            idxb = jnp.minimum(lane + b, 15)
                            cx = _lgather(x, idxb)
                            cd = _lgather(dd, idxb)
                            cm = _lgather(mm, idxb)
                            take = ((cm == 1) & ((cd & b) != 0)
                                    & (lane + b <= 15))
                            x = jnp.where(take, cx, x)
                            ndd = jnp.where(take, cd - b, dd)
                            mm = jnp.where(
                                take, 1,
                                jnp.where((dd & b) == 0, mm, 0))
                            dd = ndd
                        pends[c][pl.ds(cnts[c], 16)] = x
                        new.append(cnts[c] + ((cs15 >> (5 * c)) & 31))
                    return tuple(new)
                return lax.fori_loop(0, SEG // 16, vec_body, cnts)

            cnts = lax.fori_loop(0, NSEG, seg_body,
                                 tuple(jnp.int32(0) for _ in range(NCHUNK)))
            for c in range(NCHUNK):
                off = ((t * NCHUNK + c) * NW + w) * BCAP
                pltpu.sync_copy(pends[c], bins.at[pl.ds(off, BCAP)])
            cv = jnp.zeros((16,), jnp.int32)
            for c in range(NCHUNK):
                cv = jnp.where(lane == c, cnts[c], cv)
            cvbuf[pl.ds(0, 16)] = cv
            pltpu.sync_copy(cvbuf, counts.at[pl.ds((t * NW + w) * 16, 16)])

    return binner


GB = 3
BR = 128


def _unpack_block(cbuf, coff, fsrc, fdst, base, n, sid, toff):
    iota16 = lax.iota(jnp.int32, 16)
    for k in range(BR // 16):
        gl = base + k * 16 + iota16
        valid = gl < n
        v = cbuf[pl.ds(coff + k * 16, 16)]
        dump = CH + ((sid * 128 + k * 16 + iota16) & (DUMP - 1))
        fdst[pl.ds(k * 16, 16)] = jnp.where(valid, v & 16383, dump)
        if fsrc is not None:
            fsrc[pl.ds(k * 16, 16)] = jnp.where(
                valid, jnp.right_shift(v, 14) + toff, k * 16 + iota16)


def _consume(t, cc, scid, sid, bins, counts, cntv, flush_fn):
    chunk = scid * 3 + cc
    for pi in range(2):
        pt = sid * 2 + pi
        pltpu.sync_copy(counts.at[pl.ds((t * NW) * 16 + pt * 16, 16)], cntv)
        cvec = cntv[pl.ds(0, 16)]
        n = jnp.where(scid == 0, cvec[cc], cvec[3 + cc])
        listoff = ((t * NCHUNK + chunk) * NW + pt) * BCAP
        ngroup = (n + GB * BR - 1) // (GB * BR)

        def group_body(g, _):
            flush_fn(listoff, g * (GB * BR), n)
            return 0
        lax.fori_loop(0, ngroup, group_body, 0)


def _sc_agg():
    out_types = jax.ShapeDtypeStruct((2, NPAD, D), jnp.float32)
    scratch = (
        pltpu.VMEM_SHARED((ACC_ROWS, D), jnp.float32),
        pltpu.VMEM((16,), jnp.int32),
        pltpu.VMEM((GB * BR,), jnp.int32),
        tuple(pltpu.VMEM((BR,), jnp.int32) for _ in range(GB)),
        tuple(pltpu.VMEM((BR,), jnp.int32) for _ in range(GB)),
        tuple(pltpu.VMEM((BR, D), jnp.float32) for _ in range(GB)),
        pltpu.VMEM((ZB, D), jnp.float32),
        tuple(pltpu.SemaphoreType.DMA for _ in range(GB)),
        tuple(pltpu.SemaphoreType.DMA for _ in range(GB)),
    )
    mesh = plsc.VectorSubcoreMesh(core_axis_name="c", subcore_axis_name="s")

    @functools.partial(pl.kernel, out_type=out_types, mesh=mesh,
                       scratch_types=scratch)
    def agg(x, bins, counts, out, acc, cntv, cbuf, fsrcs, fdsts, rows,
            zb, gsems, ssems):
        scid = lax.axis_index("c")
        sid = lax.axis_index("s")
        _zero_vmem(zb, ZB, D)
        iota16 = lax.iota(jnp.int32, 16)

        for t in range(2):
            toff = t * N
            for cc in range(3):
                chunk = scid * 3 + cc

                def z_body(z, _):
                    pltpu.sync_copy(
                        zb, acc.at[pl.ds(sid * ZSPAN + z * ZB, ZB)])
                    return 0
                lax.fori_loop(0, ZSPAN // ZB, z_body, 0)
                plsc.subcore_barrier()

                for g in range(GB):
                    for k in range(BR // 16):
                        fdsts[g][pl.ds(k * 16, 16)] = (
                            CH + ((sid * 128 + k * 16 + iota16)
                                  & (DUMP - 1)))
                    pltpu.async_copy(rows[g], acc.at[fdsts[g]], ssems[g],
                                     add=True)

                def flush(listoff, base, n):
                    pltpu.sync_copy(
                        bins.at[pl.ds(listoff + base, GB * BR)], cbuf)
                    hs = []
                    for g in range(GB):
                        pltpu.make_async_copy(
                            x.at[pl.ds(0, BR)], rows[g], ssems[g]).wait()
                        _unpack_block(cbuf, g * BR, fsrcs[g], fdsts[g],
                                      base + g * BR, n, sid, toff)
                        hs.append(pltpu.async_copy(
                            x.at[fsrcs[g]], rows[g], gsems[g]))
                    for g in range(GB):
                        hs[g].wait()
                        pltpu.async_copy(rows[g], acc.at[fdsts[g]],
                                         ssems[g], add=True)

                _consume(t, cc, scid, sid, bins, counts, cntv, flush)
                for g in range(GB):
                    pltpu.make_async_copy(
                        x.at[pl.ds(0, BR)], rows[g], ssems[g]).wait()
                plsc.subcore_barrier()
                pltpu.sync_copy(
                    acc.at[pl.ds(sid * DROWS, DROWS)],
                    out.at[t].at[pl.ds(chunk * CH + sid * DROWS, DROWS)])
                plsc.subcore_barrier()

    return agg


def _sc_counts():
    out_types = jax.ShapeDtypeStruct((2, NPAD, D), jnp.float32)
    scratch = (
        pltpu.VMEM_SHARED((ACC_ROWS, D), jnp.float32),
        pltpu.VMEM((16,), jnp.int32),
        pltpu.VMEM((GB * BR,), jnp.int32),
        tuple(pltpu.VMEM((BR,), jnp.int32) for _ in range(GB)),
        pltpu.VMEM((BR, D), jnp.float32),
        pltpu.VMEM((ZB, D), jnp.float32),
    )
    mesh = plsc.VectorSubcoreMesh(core_axis_name="c", subcore_axis_name="s")

    @functools.partial(pl.kernel, out_type=out_types, mesh=mesh,
                       scratch_types=scratch)
    def ckern(bins, counts, out, acc, cntv, cbuf, fdsts, ones, zb):
        scid = lax.axis_index("c")
        sid = lax.axis_index("s")
        _zero_vmem(zb, ZB, D)
        one16 = jnp.ones((16,), jnp.float32)

        def ones_row(i, _):
            for k in range(D // 16):
                ones[i, pl.ds(k * 16, 16)] = one16
            return 0
        lax.fori_loop(0, BR, ones_row, 0)

        for t in range(2):
            for cc in range(3):
                chunk = scid * 3 + cc

                def z_body(z, _):
                    pltpu.sync_copy(
                        zb, acc.at[pl.ds(sid * ZSPAN + z * ZB, ZB)])
                    return 0
                lax.fori_loop(0, ZSPAN // ZB, z_body, 0)
                plsc.subcore_barrier()

                def flush(listoff, base, n):
                    pltpu.sync_copy(
                        bins.at[pl.ds(listoff + base, GB * BR)], cbuf)
                    for g in range(GB):
                        _unpack_block(cbuf, g * BR, None, fdsts[g],
                                      base + g * BR, n, sid, 0)
                        pltpu.sync_copy(ones, acc.at[fdsts[g]], add=True)

                _consume(t, cc, scid, sid, bins, counts, cntv, flush)
                plsc.subcore_barrier()
                pltpu.sync_copy(
                    acc.at[pl.ds(sid * DROWS, DROWS)],
                    out.at[t].at[pl.ds(chunk * CH + sid * DROWS, DROWS)])
                plsc.subcore_barrier()

    return ckern


_binner = _sc_binner()
_agg = _sc_agg()
_counts = _sc_counts()

_BLK = 2000
_GRID = N // _BLK


def _cnt_prep(cnt):
    def body(c_ref, o_ref):
        o_ref[...] = c_ref[:, :, 0:8]
    return pl.pallas_call(
        body,
        grid=(2, NPAD // 2304),
        in_specs=[pl.BlockSpec((1, 2304, D), lambda t, i: (t, i, 0))],
        out_specs=pl.BlockSpec((1, 2304, 8), lambda t, i: (t, i, 0)),
        out_shape=jax.ShapeDtypeStruct((2, NPAD, 8), jnp.float32),
    )(cnt)


def _tc_body(normalize, s_ref, c_ref, x_ref, wl_ref, wr_ref, b_ref, o_ref):
    cnt = c_ref[0, :, 0:1]
    agg = s_ref[0] / jnp.maximum(cnt, 1.0)
    out = (
        lax.dot_general(agg, wl_ref[0], (((1,), (0,)), ((), ())),
                        precision=lax.Precision.DEFAULT,
                        preferred_element_type=jnp.float32)
        + lax.dot_general(x_ref[...], wr_ref[0], (((1,), (0,)), ((), ())),
                          precision=lax.Precision.DEFAULT,
                          preferred_element_type=jnp.float32)
        + b_ref[0])
    if normalize:
        nrm = jnp.sqrt(jnp.sum(out * out, axis=1, keepdims=True))
        out = jnp.maximum(out / jnp.maximum(nrm, 1e-12), 0.0)
    o_ref[...] = out


def _tc_transform(s, cnt, x, wl, wr, bias, normalize):
    return pl.pallas_call(
        functools.partial(_tc_body, normalize),
        grid=(2, _GRID),
        in_specs=[
            pl.BlockSpec((1, _BLK, D), lambda t, i: (t, i, 0)),
            pl.BlockSpec((1, _BLK, 8), lambda t, i: (t, i, 0)),
            pl.BlockSpec((_BLK, D), lambda t, i: ((1 - t) * _GRID + i, 0)),
            pl.BlockSpec((1, D, D), lambda t, i: (t, 0, 0)),
            pl.BlockSpec((1, D, D), lambda t, i: (t, 0, 0)),
            pl.BlockSpec((1, 1, D), lambda t, i: (t, 0, 0)),
        ],
        out_specs=pl.BlockSpec((_BLK, D),
                               lambda t, i: ((1 - t) * _GRID + i, 0)),
        out_shape=jax.ShapeDtypeStruct((2 * N, D), jnp.float32),
    )(s, cnt, x, wl, wr, bias)


def kernel(x_author, x_paper, ei_writes, ei_written_by, W_l, W_r, b):
    padlen = E_PAD - E
    pad_src = jnp.arange(padlen, dtype=jnp.int32) % N
    pad_dst = jnp.full((padlen,), NPAD, jnp.int32)
    src_w = jnp.concatenate([ei_writes[0], pad_src])
    dst_w = jnp.concatenate([ei_writes[1], pad_dst])
    src_b = jnp.concatenate([ei_written_by[0], pad_src])
    dst_b = jnp.concatenate([ei_written_by[1], pad_dst])

    bins, counts = _binner(src_w, dst_w, src_b, dst_b)
    cnt = _cnt_prep(_counts(bins, counts))
    x = jnp.concatenate([x_author, x_paper], axis=0)
    for l in range(L):
        s = _agg(x, bins, counts)
        x = _tc_transform(s, cnt, x, W_l[l], W_r[l],
                          b[l].reshape(2, 1, D), l < L - 1)
    return x
